# baseline (speedup 1.0000x reference)
import functools

import jax
import jax.numpy as jnp
import numpy as np
from jax import lax
from jax.experimental import pallas as pl
from jax.experimental.pallas import tpu as pltpu
from jax.experimental.pallas import tpu_sc as plsc

D = 768
H = 1536
E = 8
T = 2048
K = 2

BT = 512
NT = T // BT
BR = 512
NB_MAX = (T * K) // BR + E - 1
NBC = NB_MAX + 1
P = NBC * BR
EPS = float(np.finfo(np.float64).eps)

_SQRT1_2 = float(1.0 / np.sqrt(2.0))


def _route_body(x_ref, c_ref, wg_ref, g_out, d0_out, d1_out, be_out,
                carry_sc, pref_sc, e12g_sc, off_sc):
    i = pl.program_id(0)
    iota8 = lax.broadcasted_iota(jnp.int32, (BT, E), 1)

    @pl.when(i < NT)
    def _pass1():
        xb = x_ref[...]
        cb = c_ref[...]
        logits = (jnp.dot(xb, wg_ref[:D, :], preferred_element_type=jnp.float32)
                  + jnp.dot(cb, wg_ref[D:, :], preferred_element_type=jnp.float32))
        m1 = jnp.max(logits, axis=1, keepdims=True)
        e1 = jnp.min(jnp.where(logits >= m1, iota8, E), axis=1, keepdims=True)
        oh1 = iota8 == e1
        neg = jnp.where(oh1, -jnp.inf, logits)
        m2 = jnp.max(neg, axis=1, keepdims=True)
        e2 = jnp.min(jnp.where(neg >= m2, iota8, E), axis=1, keepdims=True)
        oh2 = iota8 == e2
        g1 = jax.nn.sigmoid(m1 - m2)
        g2 = 1.0 - g1
        g_out[...] = jnp.concatenate([g1, g2], axis=1)

        onehot = oh1.astype(jnp.float32) + oh2.astype(jnp.float32)

        @pl.when(i == 0)
        def _():
            carry_sc[...] = jnp.zeros((1, E), jnp.float32)

        r = lax.broadcasted_iota(jnp.int32, (BT, BT), 0)
        c = lax.broadcasted_iota(jnp.int32, (BT, BT), 1)
        tri = (r > c).astype(jnp.float32)
        pref = (jnp.dot(tri, onehot, preferred_element_type=jnp.float32)
                + carry_sc[...])
        pref_sc[pl.ds(i * BT, BT), :] = pref
        e12g_sc[pl.ds(i * BT, BT), :] = jnp.concatenate(
            [e1.astype(jnp.float32), e2.astype(jnp.float32), g1, g2,
             jnp.zeros((BT, E - 4), jnp.float32)], axis=1)
        counts = carry_sc[...] + jnp.sum(onehot, axis=0, keepdims=True)
        carry_sc[...] = counts

        @pl.when(i == NT - 1)
        def _finalize():
            cblk = jnp.ceil(counts * (1.0 / BR))
            tr = lax.broadcasted_iota(jnp.int32, (E, E), 0)
            tc = lax.broadcasted_iota(jnp.int32, (E, E), 1)
            tri8 = (tr < tc).astype(jnp.float32)
            off_blk = jnp.dot(cblk, tri8, preferred_element_type=jnp.float32)
            off_sc[...] = off_blk * BR
            nb_tot = jnp.sum(cblk)
            iob = lax.broadcasted_iota(jnp.int32, (1, NBC), 1).astype(jnp.float32)
            acc = jnp.zeros((1, NBC), jnp.float32)
            for e in range(E):
                acc += (iob >= off_blk[0, e]).astype(jnp.float32)
            be = jnp.where(iob < nb_tot, acc - 1.0, -1.0)
            be_out[...] = be.astype(jnp.int32)

    @pl.when(i >= NT)
    def _pass2():
        j = i - NT
        pref = pref_sc[pl.ds(j * BT, BT), :]
        e12g = e12g_sc[pl.ds(j * BT, BT), :]
        e1 = e12g[:, 0:1].astype(jnp.int32)
        e2 = e12g[:, 1:2].astype(jnp.int32)
        off = off_sc[...]
        pos = off + pref
        oh1 = (iota8 == e1).astype(jnp.float32)
        oh2 = (iota8 == e2).astype(jnp.float32)
        d0 = jnp.sum(oh1 * pos, axis=1)
        d1 = jnp.sum(oh2 * pos, axis=1)
        d0_out[...] = d0.astype(jnp.int32)
        d1_out[...] = d1.astype(jnp.int32)


def _route(x, cond, w_gate):
    g, d0, d1, be = pl.pallas_call(
        _route_body,
        grid=(2 * NT,),
        in_specs=[
            pl.BlockSpec((BT, D), lambda i: (jnp.minimum(i, NT - 1), 0)),
            pl.BlockSpec((BT, D), lambda i: (jnp.minimum(i, NT - 1), 0)),
            pl.BlockSpec((2 * D, E), lambda i: (0, 0)),
        ],
        out_specs=[
            pl.BlockSpec((BT, K), lambda i: (jnp.minimum(i, NT - 1), 0)),
            pl.BlockSpec((BT,), lambda i: (jnp.maximum(i - NT, 0),)),
            pl.BlockSpec((BT,), lambda i: (jnp.maximum(i - NT, 0),)),
            pl.BlockSpec((1, NBC), lambda i: (0, 0)),
        ],
        out_shape=[
            jax.ShapeDtypeStruct((T, K), jnp.float32),
            jax.ShapeDtypeStruct((T,), jnp.int32),
            jax.ShapeDtypeStruct((T,), jnp.int32),
            jax.ShapeDtypeStruct((1, NBC), jnp.int32),
        ],
        scratch_shapes=[
            pltpu.VMEM((1, E), jnp.float32),
            pltpu.VMEM((T, E), jnp.float32),
            pltpu.VMEM((T, E), jnp.float32),
            pltpu.VMEM((1, E), jnp.float32),
        ],
        compiler_params=pltpu.CompilerParams(
            dimension_semantics=("arbitrary",),
        ),
    )(x, cond, w_gate)
    return g, d0, d1, be


_NC = 2
_NS = 16
_NW = _NC * _NS
_CHUNK = T // _NW


@functools.lru_cache(maxsize=None)
def _make_dispatch():
    @functools.partial(
        pl.kernel,
        out_type=jax.ShapeDtypeStruct((P, D), jnp.float32),
        mesh=plsc.VectorSubcoreMesh(core_axis_name="c", subcore_axis_name="s"),
        scratch_types=[
            pltpu.VMEM((_CHUNK,), jnp.int32),
            pltpu.VMEM((_CHUNK,), jnp.int32),
            pltpu.VMEM((_CHUNK, D), jnp.float32),
            pltpu.SemaphoreType.DMA,
            pltpu.SemaphoreType.DMA,
        ],
    )
    def _dispatch(x_hbm, d0_hbm, d1_hbm, out_hbm, i0_v, i1_v, rows_v, s0, s1):
        wid = lax.axis_index("s") * _NC + lax.axis_index("c")
        base = wid * _CHUNK
        pltpu.sync_copy(d0_hbm.at[pl.ds(base, _CHUNK)], i0_v)
        pltpu.sync_copy(d1_hbm.at[pl.ds(base, _CHUNK)], i1_v)
        pltpu.sync_copy(x_hbm.at[pl.ds(base, _CHUNK)], rows_v)
        c0 = pltpu.async_copy(rows_v, out_hbm.at[i0_v], s0)
        c1 = pltpu.async_copy(rows_v, out_hbm.at[i1_v], s1)
        c0.wait()
        c1.wait()

    return _dispatch


def _ffn_body(be_ref, x_ref, w1_ref, b1_ref, w2_ref, b2_ref, y_ref):
    b = pl.program_id(0)
    be = be_ref[0, b]

    @pl.when(be >= 0)
    def _():
        xb = x_ref[...].astype(jnp.bfloat16)
        h = (jnp.dot(xb, w1_ref[0].astype(jnp.bfloat16),
                     preferred_element_type=jnp.float32) + b1_ref[0])
        h = 0.5 * h * (1.0 + lax.erf(h * _SQRT1_2))
        o = (jnp.dot(h.astype(jnp.bfloat16), w2_ref[0].astype(jnp.bfloat16),
                     preferred_element_type=jnp.float32) + b2_ref[0])
        y_ref[...] = o


def _ffn(be, x_sorted, fc1_w, fc1_b, fc2_w, fc2_b):
    def _act(b, be_ref):
        return be_ref[0, b] >= 0

    grid_spec = pltpu.PrefetchScalarGridSpec(
        num_scalar_prefetch=1,
        grid=(NBC,),
        in_specs=[
            pl.BlockSpec((BR, D), lambda b, be: (jnp.where(_act(b, be), b, 0), 0)),
            pl.BlockSpec((1, D, H),
                         lambda b, be: (jnp.where(_act(b, be), be[0, b], 0), 0, 0)),
            pl.BlockSpec((1, 1, H),
                         lambda b, be: (jnp.where(_act(b, be), be[0, b], 0), 0, 0)),
            pl.BlockSpec((1, H, D),
                         lambda b, be: (jnp.where(_act(b, be), be[0, b], 0), 0, 0)),
            pl.BlockSpec((1, 1, D),
                         lambda b, be: (jnp.where(_act(b, be), be[0, b], 0), 0, 0)),
        ],
        out_specs=pl.BlockSpec(
            (BR, D), lambda b, be: (jnp.where(_act(b, be), b, NBC - 1), 0)),
    )
    return pl.pallas_call(
        _ffn_body,
        grid_spec=grid_spec,
        out_shape=jax.ShapeDtypeStruct((P, D), jnp.float32),
        compiler_params=pltpu.CompilerParams(
            dimension_semantics=("arbitrary",),
        ),
    )(be, x_sorted, fc1_w, fc1_b.reshape(E, 1, H), fc2_w, fc2_b.reshape(E, 1, D))


@functools.lru_cache(maxsize=None)
def _make_combine_gather():
    @functools.partial(
        pl.kernel,
        out_type=(jax.ShapeDtypeStruct((T, D), jnp.float32),
                  jax.ShapeDtypeStruct((T, D), jnp.float32)),
        mesh=plsc.VectorSubcoreMesh(core_axis_name="c", subcore_axis_name="s"),
        scratch_types=[
            pltpu.VMEM((_CHUNK,), jnp.int32),
            pltpu.VMEM((_CHUNK,), jnp.int32),
            pltpu.VMEM((_CHUNK, D), jnp.float32),
            pltpu.SemaphoreType.DMA,
        ],
    )
    def _combine_gather(y_hbm, d0_hbm, d1_hbm, a_hbm, b_hbm,
                        i0_v, i1_v, rows_v, s0):
        wid = lax.axis_index("s") * _NC + lax.axis_index("c")
        base = wid * _CHUNK
        pltpu.sync_copy(d0_hbm.at[pl.ds(base, _CHUNK)], i0_v)
        pltpu.sync_copy(d1_hbm.at[pl.ds(base, _CHUNK)], i1_v)
        pltpu.async_copy(y_hbm.at[i0_v], rows_v, s0).wait()
        pltpu.sync_copy(rows_v, a_hbm.at[pl.ds(base, _CHUNK)])
        pltpu.async_copy(y_hbm.at[i1_v], rows_v, s0).wait()
        pltpu.sync_copy(rows_v, b_hbm.at[pl.ds(base, _CHUNK)])

    return _combine_gather


def _epilogue_body(a_ref, b_ref, g_ref, o_ref):
    av = jnp.exp(a_ref[...].astype(jnp.float32))
    bv = jnp.exp(b_ref[...].astype(jnp.float32))
    c = g_ref[:, 0:1] * av + g_ref[:, 1:2] * bv
    o_ref[...] = jnp.log(jnp.where(c == 0.0, EPS, c))


def _epilogue(a, b, g):
    return pl.pallas_call(
        _epilogue_body,
        grid=(NT,),
        in_specs=[
            pl.BlockSpec((BT, D), lambda i: (i, 0)),
            pl.BlockSpec((BT, D), lambda i: (i, 0)),
            pl.BlockSpec((BT, K), lambda i: (i, 0)),
        ],
        out_specs=pl.BlockSpec((BT, D), lambda i: (i, 0)),
        out_shape=jax.ShapeDtypeStruct((T, D), jnp.float32),
        compiler_params=pltpu.CompilerParams(
            dimension_semantics=("parallel",),
        ),
    )(a, b, g)


@jax.jit
def kernel(x, cond, w_gate, fc1_w, fc1_b, fc2_w, fc2_b):
    g, d0, d1, be = _route(x, cond, w_gate)
    x_sorted = _make_dispatch()(x, d0, d1)
    y_sorted = _ffn(be, x_sorted, fc1_w, fc1_b, fc2_w, fc2_b)
    a, b = _make_combine_gather()(y_sorted, d0, d1)
    return _epilogue(a, b, g)


# speedup vs baseline: 1.1137x; 1.1137x over previous
import functools

import jax
import jax.numpy as jnp
import numpy as np
from jax import lax
from jax.experimental import pallas as pl
from jax.experimental.pallas import tpu as pltpu
from jax.experimental.pallas import tpu_sc as plsc

D = 768
H = 1536
E = 8
T = 2048
K = 2

BT = 512
NT = T // BT
BR = 512
NB_MAX = (T * K) // BR + E - 1
NBC = NB_MAX + 1
P = NBC * BR
EPS = float(np.finfo(np.float64).eps)

_SQRT1_2 = float(1.0 / np.sqrt(2.0))
D2 = D // 2


def _pack16(lo, hi):
    lo32 = lax.bitcast_convert_type(
        lo.astype(jnp.bfloat16).astype(jnp.float32), jnp.uint32) >> 16
    hi32 = lax.bitcast_convert_type(
        hi.astype(jnp.bfloat16).astype(jnp.float32), jnp.uint32) & jnp.uint32(0xFFFF0000)
    return lax.bitcast_convert_type(hi32 | lo32, jnp.float32)


def _unpack16(w):
    u = lax.bitcast_convert_type(w, jnp.uint32)
    hi = lax.bitcast_convert_type(u & jnp.uint32(0xFFFF0000), jnp.float32)
    lo = lax.bitcast_convert_type(u << 16, jnp.float32)
    return lo, hi


def _route_body(x_ref, c_ref, wg_ref, g_out, d0_out, d1_out, be_out, xp_out,
                carry_sc, pref_sc, e12g_sc, off_sc):
    i = pl.program_id(0)
    iota8 = lax.broadcasted_iota(jnp.int32, (BT, E), 1)

    @pl.when(i < NT)
    def _pass1():
        xb = x_ref[...]
        cb = c_ref[...]
        xp_out[...] = _pack16(xb[:, :D2], xb[:, D2:])
        logits = (jnp.dot(xb, wg_ref[:D, :], preferred_element_type=jnp.float32)
                  + jnp.dot(cb, wg_ref[D:, :], preferred_element_type=jnp.float32))
        m1 = jnp.max(logits, axis=1, keepdims=True)
        e1 = jnp.min(jnp.where(logits >= m1, iota8, E), axis=1, keepdims=True)
        oh1 = iota8 == e1
        neg = jnp.where(oh1, -jnp.inf, logits)
        m2 = jnp.max(neg, axis=1, keepdims=True)
        e2 = jnp.min(jnp.where(neg >= m2, iota8, E), axis=1, keepdims=True)
        oh2 = iota8 == e2
        g1 = jax.nn.sigmoid(m1 - m2)
        g2 = 1.0 - g1
        g_out[...] = jnp.concatenate([g1, g2], axis=1)

        onehot = oh1.astype(jnp.float32) + oh2.astype(jnp.float32)

        @pl.when(i == 0)
        def _():
            carry_sc[...] = jnp.zeros((1, E), jnp.float32)

        r = lax.broadcasted_iota(jnp.int32, (BT, BT), 0)
        c = lax.broadcasted_iota(jnp.int32, (BT, BT), 1)
        tri = (r > c).astype(jnp.float32)
        pref = (jnp.dot(tri, onehot, preferred_element_type=jnp.float32)
                + carry_sc[...])
        pref_sc[pl.ds(i * BT, BT), :] = pref
        e12g_sc[pl.ds(i * BT, BT), :] = jnp.concatenate(
            [e1.astype(jnp.float32), e2.astype(jnp.float32), g1, g2,
             jnp.zeros((BT, E - 4), jnp.float32)], axis=1)
        counts = carry_sc[...] + jnp.sum(onehot, axis=0, keepdims=True)
        carry_sc[...] = counts

        @pl.when(i == NT - 1)
        def _finalize():
            cblk = jnp.ceil(counts * (1.0 / BR))
            tr = lax.broadcasted_iota(jnp.int32, (E, E), 0)
            tc = lax.broadcasted_iota(jnp.int32, (E, E), 1)
            tri8 = (tr < tc).astype(jnp.float32)
            off_blk = jnp.dot(cblk, tri8, preferred_element_type=jnp.float32)
            off_sc[...] = off_blk * BR
            nb_tot = jnp.sum(cblk)
            iob = lax.broadcasted_iota(jnp.int32, (1, NBC), 1).astype(jnp.float32)
            acc = jnp.zeros((1, NBC), jnp.float32)
            for e in range(E):
                acc += (iob >= off_blk[0, e]).astype(jnp.float32)
            be = jnp.where(iob < nb_tot, acc - 1.0, -1.0)
            be_out[...] = be.astype(jnp.int32)

    @pl.when(i >= NT)
    def _pass2():
        j = i - NT
        pref = pref_sc[pl.ds(j * BT, BT), :]
        e12g = e12g_sc[pl.ds(j * BT, BT), :]
        e1 = e12g[:, 0:1].astype(jnp.int32)
        e2 = e12g[:, 1:2].astype(jnp.int32)
        off = off_sc[...]
        pos = off + pref
        oh1 = (iota8 == e1).astype(jnp.float32)
        oh2 = (iota8 == e2).astype(jnp.float32)
        d0 = jnp.sum(oh1 * pos, axis=1)
        d1 = jnp.sum(oh2 * pos, axis=1)
        d0_out[...] = d0.astype(jnp.int32)
        d1_out[...] = d1.astype(jnp.int32)


def _route(x, cond, w_gate):
    g, d0, d1, be, xp = pl.pallas_call(
        _route_body,
        grid=(2 * NT,),
        in_specs=[
            pl.BlockSpec((BT, D), lambda i: (jnp.minimum(i, NT - 1), 0)),
            pl.BlockSpec((BT, D), lambda i: (jnp.minimum(i, NT - 1), 0)),
            pl.BlockSpec((2 * D, E), lambda i: (0, 0)),
        ],
        out_specs=[
            pl.BlockSpec((BT, K), lambda i: (jnp.minimum(i, NT - 1), 0)),
            pl.BlockSpec((BT,), lambda i: (jnp.maximum(i - NT, 0),)),
            pl.BlockSpec((BT,), lambda i: (jnp.maximum(i - NT, 0),)),
            pl.BlockSpec((1, NBC), lambda i: (0, 0)),
            pl.BlockSpec((BT, D2), lambda i: (jnp.minimum(i, NT - 1), 0)),
        ],
        out_shape=[
            jax.ShapeDtypeStruct((T, K), jnp.float32),
            jax.ShapeDtypeStruct((T,), jnp.int32),
            jax.ShapeDtypeStruct((T,), jnp.int32),
            jax.ShapeDtypeStruct((1, NBC), jnp.int32),
            jax.ShapeDtypeStruct((T, D2), jnp.float32),
        ],
        scratch_shapes=[
            pltpu.VMEM((1, E), jnp.float32),
            pltpu.VMEM((T, E), jnp.float32),
            pltpu.VMEM((T, E), jnp.float32),
            pltpu.VMEM((1, E), jnp.float32),
        ],
        compiler_params=pltpu.CompilerParams(
            dimension_semantics=("arbitrary",),
        ),
    )(x, cond, w_gate)
    return g, d0, d1, be, xp


_NC = 2
_NS = 16
_NW = _NC * _NS
_CHUNK = T // _NW


@functools.lru_cache(maxsize=None)
def _make_dispatch():
    @functools.partial(
        pl.kernel,
        out_type=jax.ShapeDtypeStruct((P, D2), jnp.float32),
        mesh=plsc.VectorSubcoreMesh(core_axis_name="c", subcore_axis_name="s"),
        scratch_types=[
            pltpu.VMEM((_CHUNK,), jnp.int32),
            pltpu.VMEM((_CHUNK,), jnp.int32),
            pltpu.VMEM((_CHUNK, D2), jnp.float32),
            pltpu.SemaphoreType.DMA,
            pltpu.SemaphoreType.DMA,
        ],
    )
    def _dispatch(x_hbm, d0_hbm, d1_hbm, out_hbm, i0_v, i1_v, rows_v, s0, s1):
        wid = lax.axis_index("s") * _NC + lax.axis_index("c")
        base = wid * _CHUNK
        pltpu.sync_copy(d0_hbm.at[pl.ds(base, _CHUNK)], i0_v)
        pltpu.sync_copy(d1_hbm.at[pl.ds(base, _CHUNK)], i1_v)
        pltpu.sync_copy(x_hbm.at[pl.ds(base, _CHUNK)], rows_v)
        c0 = pltpu.async_copy(rows_v, out_hbm.at[i0_v], s0)
        c1 = pltpu.async_copy(rows_v, out_hbm.at[i1_v], s1)
        c0.wait()
        c1.wait()

    return _dispatch


def _ffn_body(be_ref, x_ref, w1_ref, b1_ref, w2_ref, b2_ref, y_ref):
    b = pl.program_id(0)
    be = be_ref[0, b]

    @pl.when(be >= 0)
    def _():
        xlo, xhi = _unpack16(x_ref[...])
        xb = jnp.concatenate([xlo, xhi], axis=1)
        h = (jnp.dot(xb, w1_ref[0], preferred_element_type=jnp.float32)
             + b1_ref[0])
        h = 0.5 * h * (1.0 + lax.erf(h * _SQRT1_2))
        o = (jnp.dot(h, w2_ref[0], preferred_element_type=jnp.float32)
             + b2_ref[0])
        y_ref[...] = _pack16(o[:, :D2], o[:, D2:])


def _ffn(be, x_sorted, fc1_w, fc1_b, fc2_w, fc2_b):
    def _act(b, be_ref):
        return be_ref[0, b] >= 0

    grid_spec = pltpu.PrefetchScalarGridSpec(
        num_scalar_prefetch=1,
        grid=(NBC,),
        in_specs=[
            pl.BlockSpec((BR, D2), lambda b, be: (jnp.where(_act(b, be), b, 0), 0)),
            pl.BlockSpec((1, D, H),
                         lambda b, be: (jnp.where(_act(b, be), be[0, b], 0), 0, 0)),
            pl.BlockSpec((1, 1, H),
                         lambda b, be: (jnp.where(_act(b, be), be[0, b], 0), 0, 0)),
            pl.BlockSpec((1, H, D),
                         lambda b, be: (jnp.where(_act(b, be), be[0, b], 0), 0, 0)),
            pl.BlockSpec((1, 1, D),
                         lambda b, be: (jnp.where(_act(b, be), be[0, b], 0), 0, 0)),
        ],
        out_specs=pl.BlockSpec(
            (BR, D2), lambda b, be: (jnp.where(_act(b, be), b, NBC - 1), 0)),
    )
    return pl.pallas_call(
        _ffn_body,
        grid_spec=grid_spec,
        out_shape=jax.ShapeDtypeStruct((P, D2), jnp.float32),
        compiler_params=pltpu.CompilerParams(
            dimension_semantics=("arbitrary",),
        ),
    )(be, x_sorted, fc1_w, fc1_b.reshape(E, 1, H), fc2_w, fc2_b.reshape(E, 1, D))


@functools.lru_cache(maxsize=None)
def _make_combine_gather():
    @functools.partial(
        pl.kernel,
        out_type=(jax.ShapeDtypeStruct((T, D2), jnp.float32),
                  jax.ShapeDtypeStruct((T, D2), jnp.float32)),
        mesh=plsc.VectorSubcoreMesh(core_axis_name="c", subcore_axis_name="s"),
        scratch_types=[
            pltpu.VMEM((_CHUNK,), jnp.int32),
            pltpu.VMEM((_CHUNK,), jnp.int32),
            pltpu.VMEM((_CHUNK, D2), jnp.float32),
            pltpu.SemaphoreType.DMA,
        ],
    )
    def _combine_gather(y_hbm, d0_hbm, d1_hbm, a_hbm, b_hbm,
                        i0_v, i1_v, rows_v, s0):
        wid = lax.axis_index("s") * _NC + lax.axis_index("c")
        base = wid * _CHUNK
        pltpu.sync_copy(d0_hbm.at[pl.ds(base, _CHUNK)], i0_v)
        pltpu.sync_copy(d1_hbm.at[pl.ds(base, _CHUNK)], i1_v)
        pltpu.async_copy(y_hbm.at[i0_v], rows_v, s0).wait()
        pltpu.sync_copy(rows_v, a_hbm.at[pl.ds(base, _CHUNK)])
        pltpu.async_copy(y_hbm.at[i1_v], rows_v, s0).wait()
        pltpu.sync_copy(rows_v, b_hbm.at[pl.ds(base, _CHUNK)])

    return _combine_gather


def _epilogue_body(a_ref, b_ref, g_ref, o_ref):
    alo, ahi = _unpack16(a_ref[...])
    blo, bhi = _unpack16(b_ref[...])
    g0 = g_ref[:, 0:1]
    g1 = g_ref[:, 1:2]
    clo = g0 * jnp.exp(alo) + g1 * jnp.exp(blo)
    chi = g0 * jnp.exp(ahi) + g1 * jnp.exp(bhi)
    o_ref[:, :D2] = jnp.log(jnp.where(clo == 0.0, EPS, clo))
    o_ref[:, D2:] = jnp.log(jnp.where(chi == 0.0, EPS, chi))


def _epilogue(a, b, g):
    return pl.pallas_call(
        _epilogue_body,
        grid=(NT,),
        in_specs=[
            pl.BlockSpec((BT, D2), lambda i: (i, 0)),
            pl.BlockSpec((BT, D2), lambda i: (i, 0)),
            pl.BlockSpec((BT, K), lambda i: (i, 0)),
        ],
        out_specs=pl.BlockSpec((BT, D), lambda i: (i, 0)),
        out_shape=jax.ShapeDtypeStruct((T, D), jnp.float32),
        compiler_params=pltpu.CompilerParams(
            dimension_semantics=("parallel",),
        ),
    )(a, b, g)


@jax.jit
def kernel(x, cond, w_gate, fc1_w, fc1_b, fc2_w, fc2_b):
    g, d0, d1, be, xp = _route(x, cond, w_gate)
    x_sorted = _make_dispatch()(xp, d0, d1)
    y_sorted = _ffn(be, x_sorted, fc1_w, fc1_b, fc2_w, fc2_b)
    a, b = _make_combine_gather()(y_sorted, d0, d1)
    return _epilogue(a, b, g)


# speedup vs baseline: 1.1145x; 1.0007x over previous
import functools

import jax
import jax.numpy as jnp
import numpy as np
from jax import lax
from jax.experimental import pallas as pl
from jax.experimental.pallas import tpu as pltpu
from jax.experimental.pallas import tpu_sc as plsc

D = 768
H = 1536
E = 8
T = 2048
K = 2

BT = 512
NT = T // BT
BR = 512
NB_MAX = (T * K) // BR + E - 1
NBC = NB_MAX + 1
P = NBC * BR
EPS = float(np.finfo(np.float64).eps)

_SQRT1_2 = float(1.0 / np.sqrt(2.0))
D2 = D // 2


def _pack16(lo, hi):
    lo32 = lax.bitcast_convert_type(
        lo.astype(jnp.bfloat16).astype(jnp.float32), jnp.uint32) >> 16
    hi32 = lax.bitcast_convert_type(
        hi.astype(jnp.bfloat16).astype(jnp.float32), jnp.uint32) & jnp.uint32(0xFFFF0000)
    return lax.bitcast_convert_type(hi32 | lo32, jnp.float32)


def _unpack16(w):
    u = lax.bitcast_convert_type(w, jnp.uint32)
    hi = lax.bitcast_convert_type(u & jnp.uint32(0xFFFF0000), jnp.float32)
    lo = lax.bitcast_convert_type(u << 16, jnp.float32)
    return lo, hi


def _route_body(x_ref, c_ref, wg_ref, g_out, d0_out, d1_out, be_out, xp_out,
                carry_sc, pref_sc, e12g_sc, off_sc):
    i = pl.program_id(0)
    iota8 = lax.broadcasted_iota(jnp.int32, (BT, E), 1)

    @pl.when(i < NT)
    def _pass1():
        xb = x_ref[...]
        cb = c_ref[...]
        xp_out[...] = _pack16(xb[:, :D2], xb[:, D2:])
        logits = (jnp.dot(xb, wg_ref[:D, :], preferred_element_type=jnp.float32)
                  + jnp.dot(cb, wg_ref[D:, :], preferred_element_type=jnp.float32))
        m1 = jnp.max(logits, axis=1, keepdims=True)
        e1 = jnp.min(jnp.where(logits >= m1, iota8, E), axis=1, keepdims=True)
        oh1 = iota8 == e1
        neg = jnp.where(oh1, -jnp.inf, logits)
        m2 = jnp.max(neg, axis=1, keepdims=True)
        e2 = jnp.min(jnp.where(neg >= m2, iota8, E), axis=1, keepdims=True)
        oh2 = iota8 == e2
        g1 = jax.nn.sigmoid(m1 - m2)
        g2 = 1.0 - g1
        g_out[...] = jnp.concatenate([g1, g2], axis=1)

        onehot = oh1.astype(jnp.float32) + oh2.astype(jnp.float32)

        @pl.when(i == 0)
        def _():
            carry_sc[...] = jnp.zeros((1, E), jnp.float32)

        r = lax.broadcasted_iota(jnp.int32, (BT, BT), 0)
        c = lax.broadcasted_iota(jnp.int32, (BT, BT), 1)
        tri = (r > c).astype(jnp.float32)
        pref = (jnp.dot(tri, onehot, preferred_element_type=jnp.float32)
                + carry_sc[...])
        pref_sc[pl.ds(i * BT, BT), :] = pref
        e12g_sc[pl.ds(i * BT, BT), :] = jnp.concatenate(
            [e1.astype(jnp.float32), e2.astype(jnp.float32), g1, g2,
             jnp.zeros((BT, E - 4), jnp.float32)], axis=1)
        counts = carry_sc[...] + jnp.sum(onehot, axis=0, keepdims=True)
        carry_sc[...] = counts

        @pl.when(i == NT - 1)
        def _finalize():
            cblk = jnp.ceil(counts * (1.0 / BR))
            tr = lax.broadcasted_iota(jnp.int32, (E, E), 0)
            tc = lax.broadcasted_iota(jnp.int32, (E, E), 1)
            tri8 = (tr < tc).astype(jnp.float32)
            off_blk = jnp.dot(cblk, tri8, preferred_element_type=jnp.float32)
            off_sc[...] = off_blk * BR
            nb_tot = jnp.sum(cblk)
            iob = lax.broadcasted_iota(jnp.int32, (1, NBC), 1).astype(jnp.float32)
            acc = jnp.zeros((1, NBC), jnp.float32)
            for e in range(E):
                acc += (iob >= off_blk[0, e]).astype(jnp.float32)
            be = jnp.where(iob < nb_tot, acc - 1.0, -1.0)
            be_out[...] = be.astype(jnp.int32)

    @pl.when(i >= NT)
    def _pass2():
        j = i - NT
        pref = pref_sc[pl.ds(j * BT, BT), :]
        e12g = e12g_sc[pl.ds(j * BT, BT), :]
        e1 = e12g[:, 0:1].astype(jnp.int32)
        e2 = e12g[:, 1:2].astype(jnp.int32)
        off = off_sc[...]
        pos = off + pref
        oh1 = (iota8 == e1).astype(jnp.float32)
        oh2 = (iota8 == e2).astype(jnp.float32)
        d0 = jnp.sum(oh1 * pos, axis=1)
        d1 = jnp.sum(oh2 * pos, axis=1)
        d0_out[...] = d0.astype(jnp.int32)
        d1_out[...] = d1.astype(jnp.int32)


def _route(x, cond, w_gate):
    g, d0, d1, be, xp = pl.pallas_call(
        _route_body,
        grid=(2 * NT,),
        in_specs=[
            pl.BlockSpec((BT, D), lambda i: (jnp.minimum(i, NT - 1), 0)),
            pl.BlockSpec((BT, D), lambda i: (jnp.minimum(i, NT - 1), 0)),
            pl.BlockSpec((2 * D, E), lambda i: (0, 0)),
        ],
        out_specs=[
            pl.BlockSpec((BT, K), lambda i: (jnp.minimum(i, NT - 1), 0)),
            pl.BlockSpec((BT,), lambda i: (jnp.maximum(i - NT, 0),)),
            pl.BlockSpec((BT,), lambda i: (jnp.maximum(i - NT, 0),)),
            pl.BlockSpec((1, NBC), lambda i: (0, 0)),
            pl.BlockSpec((BT, D2), lambda i: (jnp.minimum(i, NT - 1), 0)),
        ],
        out_shape=[
            jax.ShapeDtypeStruct((T, K), jnp.float32),
            jax.ShapeDtypeStruct((T,), jnp.int32),
            jax.ShapeDtypeStruct((T,), jnp.int32),
            jax.ShapeDtypeStruct((1, NBC), jnp.int32),
            jax.ShapeDtypeStruct((T, D2), jnp.float32),
        ],
        scratch_shapes=[
            pltpu.VMEM((1, E), jnp.float32),
            pltpu.VMEM((T, E), jnp.float32),
            pltpu.VMEM((T, E), jnp.float32),
            pltpu.VMEM((1, E), jnp.float32),
        ],
        compiler_params=pltpu.CompilerParams(
            dimension_semantics=("arbitrary",),
        ),
    )(x, cond, w_gate)
    return g, d0, d1, be, xp


_NC = 2
_NS = 16
_NW = _NC * _NS
_CHUNK = T // _NW


@functools.lru_cache(maxsize=None)
def _make_dispatch():
    @functools.partial(
        pl.kernel,
        out_type=jax.ShapeDtypeStruct((P, D2), jnp.float32),
        mesh=plsc.VectorSubcoreMesh(core_axis_name="c", subcore_axis_name="s"),
        scratch_types=[
            pltpu.VMEM((_CHUNK,), jnp.int32),
            pltpu.VMEM((_CHUNK,), jnp.int32),
            pltpu.VMEM((_CHUNK, D2), jnp.float32),
            pltpu.SemaphoreType.DMA,
            pltpu.SemaphoreType.DMA,
        ],
    )
    def _dispatch(x_hbm, d0_hbm, d1_hbm, out_hbm, i0_v, i1_v, rows_v, s0, s1):
        wid = lax.axis_index("s") * _NC + lax.axis_index("c")
        base = wid * _CHUNK
        pltpu.sync_copy(d0_hbm.at[pl.ds(base, _CHUNK)], i0_v)
        pltpu.sync_copy(d1_hbm.at[pl.ds(base, _CHUNK)], i1_v)
        pltpu.sync_copy(x_hbm.at[pl.ds(base, _CHUNK)], rows_v)
        c0 = pltpu.async_copy(rows_v, out_hbm.at[i0_v], s0)
        c1 = pltpu.async_copy(rows_v, out_hbm.at[i1_v], s1)
        c0.wait()
        c1.wait()

    return _dispatch


def _ffn_body(be_ref, x_ref, w1_ref, b1_ref, w2_ref, b2_ref, y_ref):
    b = pl.program_id(0)
    be = be_ref[0, b]

    @pl.when(be >= 0)
    def _():
        xlo, xhi = _unpack16(x_ref[...])
        xb = jnp.concatenate([xlo, xhi], axis=1)
        h = (jnp.dot(xb, w1_ref[0], preferred_element_type=jnp.float32,
                     precision=lax.Precision.DEFAULT) + b1_ref[0])
        h = 0.5 * h * (1.0 + lax.erf(h * _SQRT1_2))
        o = (jnp.dot(h, w2_ref[0], preferred_element_type=jnp.float32,
                     precision=lax.Precision.DEFAULT) + b2_ref[0])
        y_ref[...] = _pack16(o[:, :D2], o[:, D2:])


def _ffn(be, x_sorted, fc1_w, fc1_b, fc2_w, fc2_b):
    def _act(b, be_ref):
        return be_ref[0, b] >= 0

    grid_spec = pltpu.PrefetchScalarGridSpec(
        num_scalar_prefetch=1,
        grid=(NBC,),
        in_specs=[
            pl.BlockSpec((BR, D2), lambda b, be: (jnp.where(_act(b, be), b, 0), 0)),
            pl.BlockSpec((1, D, H),
                         lambda b, be: (jnp.where(_act(b, be), be[0, b], 0), 0, 0)),
            pl.BlockSpec((1, 1, H),
                         lambda b, be: (jnp.where(_act(b, be), be[0, b], 0), 0, 0)),
            pl.BlockSpec((1, H, D),
                         lambda b, be: (jnp.where(_act(b, be), be[0, b], 0), 0, 0)),
            pl.BlockSpec((1, 1, D),
                         lambda b, be: (jnp.where(_act(b, be), be[0, b], 0), 0, 0)),
        ],
        out_specs=pl.BlockSpec(
            (BR, D2), lambda b, be: (jnp.where(_act(b, be), b, NBC - 1), 0)),
    )
    return pl.pallas_call(
        _ffn_body,
        grid_spec=grid_spec,
        out_shape=jax.ShapeDtypeStruct((P, D2), jnp.float32),
        compiler_params=pltpu.CompilerParams(
            dimension_semantics=("arbitrary",),
        ),
    )(be, x_sorted, fc1_w, fc1_b.reshape(E, 1, H), fc2_w, fc2_b.reshape(E, 1, D))


@functools.lru_cache(maxsize=None)
def _make_combine_gather():
    @functools.partial(
        pl.kernel,
        out_type=(jax.ShapeDtypeStruct((T, D2), jnp.float32),
                  jax.ShapeDtypeStruct((T, D2), jnp.float32)),
        mesh=plsc.VectorSubcoreMesh(core_axis_name="c", subcore_axis_name="s"),
        scratch_types=[
            pltpu.VMEM((_CHUNK,), jnp.int32),
            pltpu.VMEM((_CHUNK,), jnp.int32),
            pltpu.VMEM((_CHUNK, D2), jnp.float32),
            pltpu.SemaphoreType.DMA,
        ],
    )
    def _combine_gather(y_hbm, d0_hbm, d1_hbm, a_hbm, b_hbm,
                        i0_v, i1_v, rows_v, s0):
        wid = lax.axis_index("s") * _NC + lax.axis_index("c")
        base = wid * _CHUNK
        pltpu.sync_copy(d0_hbm.at[pl.ds(base, _CHUNK)], i0_v)
        pltpu.sync_copy(d1_hbm.at[pl.ds(base, _CHUNK)], i1_v)
        pltpu.async_copy(y_hbm.at[i0_v], rows_v, s0).wait()
        pltpu.sync_copy(rows_v, a_hbm.at[pl.ds(base, _CHUNK)])
        pltpu.async_copy(y_hbm.at[i1_v], rows_v, s0).wait()
        pltpu.sync_copy(rows_v, b_hbm.at[pl.ds(base, _CHUNK)])

    return _combine_gather


def _epilogue_body(a_ref, b_ref, g_ref, o_ref):
    alo, ahi = _unpack16(a_ref[...])
    blo, bhi = _unpack16(b_ref[...])
    g0 = g_ref[:, 0:1]
    g1 = g_ref[:, 1:2]
    clo = g0 * jnp.exp(alo) + g1 * jnp.exp(blo)
    chi = g0 * jnp.exp(ahi) + g1 * jnp.exp(bhi)
    o_ref[:, :D2] = jnp.log(jnp.where(clo == 0.0, EPS, clo))
    o_ref[:, D2:] = jnp.log(jnp.where(chi == 0.0, EPS, chi))


def _epilogue(a, b, g):
    return pl.pallas_call(
        _epilogue_body,
        grid=(NT,),
        in_specs=[
            pl.BlockSpec((BT, D2), lambda i: (i, 0)),
            pl.BlockSpec((BT, D2), lambda i: (i, 0)),
            pl.BlockSpec((BT, K), lambda i: (i, 0)),
        ],
        out_specs=pl.BlockSpec((BT, D), lambda i: (i, 0)),
        out_shape=jax.ShapeDtypeStruct((T, D), jnp.float32),
        compiler_params=pltpu.CompilerParams(
            dimension_semantics=("parallel",),
        ),
    )(a, b, g)


@jax.jit
def kernel(x, cond, w_gate, fc1_w, fc1_b, fc2_w, fc2_b):
    g, d0, d1, be, xp = _route(x, cond, w_gate)
    x_sorted = _make_dispatch()(xp, d0, d1)
    y_sorted = _ffn(be, x_sorted, fc1_w, fc1_b, fc2_w, fc2_b)
    a, b = _make_combine_gather()(y_sorted, d0, d1)
    return _epilogue(a, b, g)


# speedup vs baseline: 1.1224x; 1.0071x over previous
import functools

import jax
import jax.numpy as jnp
import numpy as np
from jax import lax
from jax.experimental import pallas as pl
from jax.experimental.pallas import tpu as pltpu
from jax.experimental.pallas import tpu_sc as plsc

D = 768
H = 1536
E = 8
T = 2048
K = 2

BT = 1024
NT = T // BT
BE = 512
NE = T // BE
BR = 512
NB_MAX = (T * K) // BR + E - 1
NBC = NB_MAX + 1
P = NBC * BR
EPS = float(np.finfo(np.float64).eps)

_SQRT1_2 = float(1.0 / np.sqrt(2.0))
D2 = D // 2


def _pack16(lo, hi):
    lo32 = lax.bitcast_convert_type(
        lo.astype(jnp.bfloat16).astype(jnp.float32), jnp.uint32) >> 16
    hi32 = lax.bitcast_convert_type(
        hi.astype(jnp.bfloat16).astype(jnp.float32), jnp.uint32) & jnp.uint32(0xFFFF0000)
    return lax.bitcast_convert_type(hi32 | lo32, jnp.float32)


def _unpack16(w):
    u = lax.bitcast_convert_type(w, jnp.uint32)
    hi = lax.bitcast_convert_type(u & jnp.uint32(0xFFFF0000), jnp.float32)
    lo = lax.bitcast_convert_type(u << 16, jnp.float32)
    return lo, hi


def _route_body(x_ref, c_ref, wg_ref, g_out, d0_out, d1_out, be_out, xp_out,
                carry_sc, pref_sc, e12g_sc, off_sc):
    i = pl.program_id(0)
    iota8 = lax.broadcasted_iota(jnp.int32, (BT, E), 1)

    @pl.when(i < NT)
    def _pass1():
        xb = x_ref[...]
        cb = c_ref[...]
        xp_out[...] = _pack16(xb[:, :D2], xb[:, D2:])
        logits = (jnp.dot(xb, wg_ref[:D, :], preferred_element_type=jnp.float32)
                  + jnp.dot(cb, wg_ref[D:, :], preferred_element_type=jnp.float32))
        m1 = jnp.max(logits, axis=1, keepdims=True)
        e1 = jnp.min(jnp.where(logits >= m1, iota8, E), axis=1, keepdims=True)
        oh1 = iota8 == e1
        neg = jnp.where(oh1, -jnp.inf, logits)
        m2 = jnp.max(neg, axis=1, keepdims=True)
        e2 = jnp.min(jnp.where(neg >= m2, iota8, E), axis=1, keepdims=True)
        oh2 = iota8 == e2
        g1 = jax.nn.sigmoid(m1 - m2)
        g2 = 1.0 - g1
        g_out[...] = jnp.concatenate([g1, g2], axis=1)

        onehot = oh1.astype(jnp.float32) + oh2.astype(jnp.float32)

        @pl.when(i == 0)
        def _():
            carry_sc[...] = jnp.zeros((1, E), jnp.float32)

        r = lax.broadcasted_iota(jnp.int32, (BT, BT), 0)
        c = lax.broadcasted_iota(jnp.int32, (BT, BT), 1)
        tri = (r > c).astype(jnp.float32)
        pref = (jnp.dot(tri, onehot, preferred_element_type=jnp.float32)
                + carry_sc[...])
        pref_sc[pl.ds(i * BT, BT), :] = pref
        e12g_sc[pl.ds(i * BT, BT), :] = jnp.concatenate(
            [e1.astype(jnp.float32), e2.astype(jnp.float32), g1, g2,
             jnp.zeros((BT, E - 4), jnp.float32)], axis=1)
        counts = carry_sc[...] + jnp.sum(onehot, axis=0, keepdims=True)
        carry_sc[...] = counts

        @pl.when(i == NT - 1)
        def _finalize():
            cblk = jnp.ceil(counts * (1.0 / BR))
            tr = lax.broadcasted_iota(jnp.int32, (E, E), 0)
            tc = lax.broadcasted_iota(jnp.int32, (E, E), 1)
            tri8 = (tr < tc).astype(jnp.float32)
            off_blk = jnp.dot(cblk, tri8, preferred_element_type=jnp.float32)
            off_sc[...] = off_blk * BR
            nb_tot = jnp.sum(cblk)
            iob = lax.broadcasted_iota(jnp.int32, (1, NBC), 1).astype(jnp.float32)
            acc = jnp.zeros((1, NBC), jnp.float32)
            for e in range(E):
                acc += (iob >= off_blk[0, e]).astype(jnp.float32)
            be = jnp.where(iob < nb_tot, acc - 1.0, -1.0)
            be_out[...] = be.astype(jnp.int32)

    @pl.when(i >= NT)
    def _pass2():
        j = i - NT
        pref = pref_sc[pl.ds(j * BT, BT), :]
        e12g = e12g_sc[pl.ds(j * BT, BT), :]
        e1 = e12g[:, 0:1].astype(jnp.int32)
        e2 = e12g[:, 1:2].astype(jnp.int32)
        off = off_sc[...]
        pos = off + pref
        oh1 = (iota8 == e1).astype(jnp.float32)
        oh2 = (iota8 == e2).astype(jnp.float32)
        d0 = jnp.sum(oh1 * pos, axis=1)
        d1 = jnp.sum(oh2 * pos, axis=1)
        d0_out[...] = d0.astype(jnp.int32)
        d1_out[...] = d1.astype(jnp.int32)


def _route(x, cond, w_gate):
    g, d0, d1, be, xp = pl.pallas_call(
        _route_body,
        grid=(2 * NT,),
        in_specs=[
            pl.BlockSpec((BT, D), lambda i: (jnp.minimum(i, NT - 1), 0)),
            pl.BlockSpec((BT, D), lambda i: (jnp.minimum(i, NT - 1), 0)),
            pl.BlockSpec((2 * D, E), lambda i: (0, 0)),
        ],
        out_specs=[
            pl.BlockSpec((BT, K), lambda i: (jnp.minimum(i, NT - 1), 0)),
            pl.BlockSpec((BT,), lambda i: (jnp.maximum(i - NT, 0),)),
            pl.BlockSpec((BT,), lambda i: (jnp.maximum(i - NT, 0),)),
            pl.BlockSpec((1, NBC), lambda i: (0, 0)),
            pl.BlockSpec((BT, D2), lambda i: (jnp.minimum(i, NT - 1), 0)),
        ],
        out_shape=[
            jax.ShapeDtypeStruct((T, K), jnp.float32),
            jax.ShapeDtypeStruct((T,), jnp.int32),
            jax.ShapeDtypeStruct((T,), jnp.int32),
            jax.ShapeDtypeStruct((1, NBC), jnp.int32),
            jax.ShapeDtypeStruct((T, D2), jnp.float32),
        ],
        scratch_shapes=[
            pltpu.VMEM((1, E), jnp.float32),
            pltpu.VMEM((T, E), jnp.float32),
            pltpu.VMEM((T, E), jnp.float32),
            pltpu.VMEM((1, E), jnp.float32),
        ],
        compiler_params=pltpu.CompilerParams(
            dimension_semantics=("arbitrary",),
        ),
    )(x, cond, w_gate)
    return g, d0, d1, be, xp


_NC = 2
_NS = 16
_NW = _NC * _NS
_CHUNK = T // _NW


@functools.lru_cache(maxsize=None)
def _make_dispatch():
    @functools.partial(
        pl.kernel,
        out_type=jax.ShapeDtypeStruct((P, D2), jnp.float32),
        mesh=plsc.VectorSubcoreMesh(core_axis_name="c", subcore_axis_name="s"),
        scratch_types=[
            pltpu.VMEM((_CHUNK,), jnp.int32),
            pltpu.VMEM((_CHUNK,), jnp.int32),
            pltpu.VMEM((_CHUNK, D2), jnp.float32),
            pltpu.SemaphoreType.DMA,
            pltpu.SemaphoreType.DMA,
        ],
    )
    def _dispatch(x_hbm, d0_hbm, d1_hbm, out_hbm, i0_v, i1_v, rows_v, s0, s1):
        wid = lax.axis_index("s") * _NC + lax.axis_index("c")
        base = wid * _CHUNK
        pltpu.sync_copy(d0_hbm.at[pl.ds(base, _CHUNK)], i0_v)
        pltpu.sync_copy(d1_hbm.at[pl.ds(base, _CHUNK)], i1_v)
        pltpu.sync_copy(x_hbm.at[pl.ds(base, _CHUNK)], rows_v)
        c0 = pltpu.async_copy(rows_v, out_hbm.at[i0_v], s0)
        c1 = pltpu.async_copy(rows_v, out_hbm.at[i1_v], s1)
        c0.wait()
        c1.wait()

    return _dispatch


def _ffn_body(be_ref, x_ref, w1_ref, b1_ref, w2_ref, b2_ref, y_ref):
    b = pl.program_id(0)
    be = be_ref[0, b]

    @pl.when(be >= 0)
    def _():
        xlo, xhi = _unpack16(x_ref[...])
        xb = jnp.concatenate([xlo, xhi], axis=1)
        h = (jnp.dot(xb, w1_ref[0], preferred_element_type=jnp.float32,
                     precision=lax.Precision.DEFAULT) + b1_ref[0])
        h = 0.5 * h * (1.0 + lax.erf(h * _SQRT1_2))
        o = (jnp.dot(h, w2_ref[0], preferred_element_type=jnp.float32,
                     precision=lax.Precision.DEFAULT) + b2_ref[0])
        y_ref[...] = _pack16(o[:, :D2], o[:, D2:])


def _ffn(be, x_sorted, fc1_w, fc1_b, fc2_w, fc2_b):
    def _act(b, be_ref):
        return be_ref[0, b] >= 0

    grid_spec = pltpu.PrefetchScalarGridSpec(
        num_scalar_prefetch=1,
        grid=(NBC,),
        in_specs=[
            pl.BlockSpec((BR, D2), lambda b, be: (jnp.where(_act(b, be), b, 0), 0)),
            pl.BlockSpec((1, D, H),
                         lambda b, be: (jnp.where(_act(b, be), be[0, b], 0), 0, 0)),
            pl.BlockSpec((1, 1, H),
                         lambda b, be: (jnp.where(_act(b, be), be[0, b], 0), 0, 0)),
            pl.BlockSpec((1, H, D),
                         lambda b, be: (jnp.where(_act(b, be), be[0, b], 0), 0, 0)),
            pl.BlockSpec((1, 1, D),
                         lambda b, be: (jnp.where(_act(b, be), be[0, b], 0), 0, 0)),
        ],
        out_specs=pl.BlockSpec(
            (BR, D2), lambda b, be: (jnp.where(_act(b, be), b, NBC - 1), 0)),
    )
    return pl.pallas_call(
        _ffn_body,
        grid_spec=grid_spec,
        out_shape=jax.ShapeDtypeStruct((P, D2), jnp.float32),
        compiler_params=pltpu.CompilerParams(
            dimension_semantics=("arbitrary",),
        ),
    )(be, x_sorted, fc1_w, fc1_b.reshape(E, 1, H), fc2_w, fc2_b.reshape(E, 1, D))


@functools.lru_cache(maxsize=None)
def _make_combine_gather():
    @functools.partial(
        pl.kernel,
        out_type=(jax.ShapeDtypeStruct((T, D2), jnp.float32),
                  jax.ShapeDtypeStruct((T, D2), jnp.float32)),
        mesh=plsc.VectorSubcoreMesh(core_axis_name="c", subcore_axis_name="s"),
        scratch_types=[
            pltpu.VMEM((_CHUNK,), jnp.int32),
            pltpu.VMEM((_CHUNK,), jnp.int32),
            pltpu.VMEM((_CHUNK, D2), jnp.float32),
            pltpu.VMEM((_CHUNK, D2), jnp.float32),
            pltpu.SemaphoreType.DMA,
            pltpu.SemaphoreType.DMA,
        ],
    )
    def _combine_gather(y_hbm, d0_hbm, d1_hbm, a_hbm, b_hbm,
                        i0_v, i1_v, ra_v, rb_v, s0, s1):
        wid = lax.axis_index("s") * _NC + lax.axis_index("c")
        base = wid * _CHUNK
        pltpu.sync_copy(d0_hbm.at[pl.ds(base, _CHUNK)], i0_v)
        pltpu.sync_copy(d1_hbm.at[pl.ds(base, _CHUNK)], i1_v)
        c0 = pltpu.async_copy(y_hbm.at[i0_v], ra_v, s0)
        c1 = pltpu.async_copy(y_hbm.at[i1_v], rb_v, s1)
        c0.wait()
        pltpu.sync_copy(ra_v, a_hbm.at[pl.ds(base, _CHUNK)])
        c1.wait()
        pltpu.sync_copy(rb_v, b_hbm.at[pl.ds(base, _CHUNK)])

    return _combine_gather


def _epilogue_body(a_ref, b_ref, g_ref, o_ref):
    alo, ahi = _unpack16(a_ref[...])
    blo, bhi = _unpack16(b_ref[...])
    g0 = g_ref[:, 0:1]
    g1 = g_ref[:, 1:2]
    clo = g0 * jnp.exp(alo) + g1 * jnp.exp(blo)
    chi = g0 * jnp.exp(ahi) + g1 * jnp.exp(bhi)
    o_ref[:, :D2] = jnp.log(jnp.where(clo == 0.0, EPS, clo))
    o_ref[:, D2:] = jnp.log(jnp.where(chi == 0.0, EPS, chi))


def _epilogue(a, b, g):
    return pl.pallas_call(
        _epilogue_body,
        grid=(NE,),
        in_specs=[
            pl.BlockSpec((BE, D2), lambda i: (i, 0)),
            pl.BlockSpec((BE, D2), lambda i: (i, 0)),
            pl.BlockSpec((BE, K), lambda i: (i, 0)),
        ],
        out_specs=pl.BlockSpec((BE, D), lambda i: (i, 0)),
        out_shape=jax.ShapeDtypeStruct((T, D), jnp.float32),
        compiler_params=pltpu.CompilerParams(
            dimension_semantics=("parallel",),
        ),
    )(a, b, g)


@jax.jit
def kernel(x, cond, w_gate, fc1_w, fc1_b, fc2_w, fc2_b):
    g, d0, d1, be, xp = _route(x, cond, w_gate)
    x_sorted = _make_dispatch()(xp, d0, d1)
    y_sorted = _ffn(be, x_sorted, fc1_w, fc1_b, fc2_w, fc2_b)
    a, b = _make_combine_gather()(y_sorted, d0, d1)
    return _epilogue(a, b, g)


# speedup vs baseline: 1.1507x; 1.0252x over previous
import functools

import jax
import jax.numpy as jnp
import numpy as np
from jax import lax
from jax.experimental import pallas as pl
from jax.experimental.pallas import tpu as pltpu
from jax.experimental.pallas import tpu_sc as plsc

D = 768
H = 1536
E = 8
T = 2048
K = 2

BT = 1024
NT = T // BT
BE = 512
NE = T // BE
BR = 512
NB_MAX = (T * K) // BR + E - 1
NBC = NB_MAX + 1
P = NBC * BR
EPS = float(np.finfo(np.float64).eps)

_SQRT1_2 = float(1.0 / np.sqrt(2.0))
D2 = D // 2


def _pack16(lo, hi):
    lo32 = lax.bitcast_convert_type(
        lo.astype(jnp.bfloat16).astype(jnp.float32), jnp.uint32) >> 16
    hi32 = lax.bitcast_convert_type(
        hi.astype(jnp.bfloat16).astype(jnp.float32), jnp.uint32) & jnp.uint32(0xFFFF0000)
    return lax.bitcast_convert_type(hi32 | lo32, jnp.float32)


def _unpack16(w):
    u = lax.bitcast_convert_type(w, jnp.uint32)
    hi = lax.bitcast_convert_type(u & jnp.uint32(0xFFFF0000), jnp.float32)
    lo = lax.bitcast_convert_type(u << 16, jnp.float32)
    return lo, hi


def _route_body(x_ref, c_ref, wg_ref, g_out, d0_out, d1_out, be_out, xp_out,
                carry_sc, pref_sc, e12g_sc, off_sc):
    i = pl.program_id(0)
    iota8 = lax.broadcasted_iota(jnp.int32, (BT, E), 1)

    @pl.when(i < NT)
    def _pass1():
        xb = x_ref[...]
        cb = c_ref[...]
        xp_out[...] = _pack16(xb[:, :D2], xb[:, D2:])
        logits = (jnp.dot(xb, wg_ref[:D, :], preferred_element_type=jnp.float32)
                  + jnp.dot(cb, wg_ref[D:, :], preferred_element_type=jnp.float32))
        m1 = jnp.max(logits, axis=1, keepdims=True)
        e1 = jnp.min(jnp.where(logits >= m1, iota8, E), axis=1, keepdims=True)
        oh1 = iota8 == e1
        neg = jnp.where(oh1, -jnp.inf, logits)
        m2 = jnp.max(neg, axis=1, keepdims=True)
        e2 = jnp.min(jnp.where(neg >= m2, iota8, E), axis=1, keepdims=True)
        oh2 = iota8 == e2
        g1 = jax.nn.sigmoid(m1 - m2)
        g2 = 1.0 - g1
        g_out[...] = jnp.concatenate([g1, g2], axis=1)

        onehot = oh1.astype(jnp.float32) + oh2.astype(jnp.float32)

        @pl.when(i == 0)
        def _():
            carry_sc[...] = jnp.zeros((1, E), jnp.float32)

        inc = onehot
        sh = 1
        while sh < BT:
            inc = inc + jnp.concatenate(
                [jnp.zeros((sh, E), jnp.float32), inc[:BT - sh]], axis=0)
            sh *= 2
        pref = (inc - onehot) + carry_sc[...]
        pref_sc[pl.ds(i * BT, BT), :] = pref
        e12g_sc[pl.ds(i * BT, BT), :] = jnp.concatenate(
            [e1.astype(jnp.float32), e2.astype(jnp.float32), g1, g2,
             jnp.zeros((BT, E - 4), jnp.float32)], axis=1)
        counts = carry_sc[...] + jnp.sum(onehot, axis=0, keepdims=True)
        carry_sc[...] = counts

        @pl.when(i == NT - 1)
        def _finalize():
            cblk = jnp.ceil(counts * (1.0 / BR))
            tr = lax.broadcasted_iota(jnp.int32, (E, E), 0)
            tc = lax.broadcasted_iota(jnp.int32, (E, E), 1)
            tri8 = (tr < tc).astype(jnp.float32)
            off_blk = jnp.dot(cblk, tri8, preferred_element_type=jnp.float32)
            off_sc[...] = off_blk * BR
            nb_tot = jnp.sum(cblk)
            iob = lax.broadcasted_iota(jnp.int32, (1, NBC), 1).astype(jnp.float32)
            acc = jnp.zeros((1, NBC), jnp.float32)
            for e in range(E):
                acc += (iob >= off_blk[0, e]).astype(jnp.float32)
            be = jnp.where(iob < nb_tot, acc - 1.0, -1.0)
            be_out[...] = be.astype(jnp.int32)

    @pl.when(i >= NT)
    def _pass2():
        j = i - NT
        pref = pref_sc[pl.ds(j * BT, BT), :]
        e12g = e12g_sc[pl.ds(j * BT, BT), :]
        e1 = e12g[:, 0:1].astype(jnp.int32)
        e2 = e12g[:, 1:2].astype(jnp.int32)
        off = off_sc[...]
        pos = off + pref
        oh1 = (iota8 == e1).astype(jnp.float32)
        oh2 = (iota8 == e2).astype(jnp.float32)
        d0 = jnp.sum(oh1 * pos, axis=1)
        d1 = jnp.sum(oh2 * pos, axis=1)
        d0_out[...] = d0.astype(jnp.int32)
        d1_out[...] = d1.astype(jnp.int32)


def _route(x, cond, w_gate):
    g, d0, d1, be, xp = pl.pallas_call(
        _route_body,
        grid=(2 * NT,),
        in_specs=[
            pl.BlockSpec((BT, D), lambda i: (jnp.minimum(i, NT - 1), 0)),
            pl.BlockSpec((BT, D), lambda i: (jnp.minimum(i, NT - 1), 0)),
            pl.BlockSpec((2 * D, E), lambda i: (0, 0)),
        ],
        out_specs=[
            pl.BlockSpec((BT, K), lambda i: (jnp.minimum(i, NT - 1), 0)),
            pl.BlockSpec((BT,), lambda i: (jnp.maximum(i - NT, 0),)),
            pl.BlockSpec((BT,), lambda i: (jnp.maximum(i - NT, 0),)),
            pl.BlockSpec((1, NBC), lambda i: (0, 0)),
            pl.BlockSpec((BT, D2), lambda i: (jnp.minimum(i, NT - 1), 0)),
        ],
        out_shape=[
            jax.ShapeDtypeStruct((T, K), jnp.float32),
            jax.ShapeDtypeStruct((T,), jnp.int32),
            jax.ShapeDtypeStruct((T,), jnp.int32),
            jax.ShapeDtypeStruct((1, NBC), jnp.int32),
            jax.ShapeDtypeStruct((T, D2), jnp.float32),
        ],
        scratch_shapes=[
            pltpu.VMEM((1, E), jnp.float32),
            pltpu.VMEM((T, E), jnp.float32),
            pltpu.VMEM((T, E), jnp.float32),
            pltpu.VMEM((1, E), jnp.float32),
        ],
        compiler_params=pltpu.CompilerParams(
            dimension_semantics=("arbitrary",),
        ),
    )(x, cond, w_gate)
    return g, d0, d1, be, xp


_NC = 2
_NS = 16
_NW = _NC * _NS
_CHUNK = T // _NW


@functools.lru_cache(maxsize=None)
def _make_dispatch():
    @functools.partial(
        pl.kernel,
        out_type=jax.ShapeDtypeStruct((P, D2), jnp.float32),
        mesh=plsc.VectorSubcoreMesh(core_axis_name="c", subcore_axis_name="s"),
        scratch_types=[
            pltpu.VMEM((_CHUNK,), jnp.int32),
            pltpu.VMEM((_CHUNK,), jnp.int32),
            pltpu.VMEM((_CHUNK, D2), jnp.float32),
            pltpu.SemaphoreType.DMA,
            pltpu.SemaphoreType.DMA,
        ],
    )
    def _dispatch(x_hbm, d0_hbm, d1_hbm, out_hbm, i0_v, i1_v, rows_v, s0, s1):
        wid = lax.axis_index("s") * _NC + lax.axis_index("c")
        base = wid * _CHUNK
        pltpu.sync_copy(d0_hbm.at[pl.ds(base, _CHUNK)], i0_v)
        pltpu.sync_copy(d1_hbm.at[pl.ds(base, _CHUNK)], i1_v)
        pltpu.sync_copy(x_hbm.at[pl.ds(base, _CHUNK)], rows_v)
        c0 = pltpu.async_copy(rows_v, out_hbm.at[i0_v], s0)
        c1 = pltpu.async_copy(rows_v, out_hbm.at[i1_v], s1)
        c0.wait()
        c1.wait()

    return _dispatch


def _ffn_body(be_ref, x_ref, w1_ref, b1_ref, w2_ref, b2_ref, y_ref):
    b = pl.program_id(0)
    be = be_ref[0, b]

    @pl.when(be >= 0)
    def _():
        xlo, xhi = _unpack16(x_ref[...])
        xb = jnp.concatenate([xlo, xhi], axis=1)
        h = (jnp.dot(xb, w1_ref[0], preferred_element_type=jnp.float32,
                     precision=lax.Precision.DEFAULT) + b1_ref[0])
        h = 0.5 * h * (1.0 + lax.erf(h * _SQRT1_2))
        o = (jnp.dot(h, w2_ref[0], preferred_element_type=jnp.float32,
                     precision=lax.Precision.DEFAULT) + b2_ref[0])
        y_ref[...] = _pack16(o[:, :D2], o[:, D2:])


def _ffn(be, x_sorted, fc1_w, fc1_b, fc2_w, fc2_b):
    def _act(b, be_ref):
        return be_ref[0, b] >= 0

    grid_spec = pltpu.PrefetchScalarGridSpec(
        num_scalar_prefetch=1,
        grid=(NBC,),
        in_specs=[
            pl.BlockSpec((BR, D2), lambda b, be: (jnp.where(_act(b, be), b, 0), 0)),
            pl.BlockSpec((1, D, H),
                         lambda b, be: (jnp.where(_act(b, be), be[0, b], 0), 0, 0)),
            pl.BlockSpec((1, 1, H),
                         lambda b, be: (jnp.where(_act(b, be), be[0, b], 0), 0, 0)),
            pl.BlockSpec((1, H, D),
                         lambda b, be: (jnp.where(_act(b, be), be[0, b], 0), 0, 0)),
            pl.BlockSpec((1, 1, D),
                         lambda b, be: (jnp.where(_act(b, be), be[0, b], 0), 0, 0)),
        ],
        out_specs=pl.BlockSpec(
            (BR, D2), lambda b, be: (jnp.where(_act(b, be), b, NBC - 1), 0)),
    )
    return pl.pallas_call(
        _ffn_body,
        grid_spec=grid_spec,
        out_shape=jax.ShapeDtypeStruct((P, D2), jnp.float32),
        compiler_params=pltpu.CompilerParams(
            dimension_semantics=("arbitrary",),
        ),
    )(be, x_sorted, fc1_w, fc1_b.reshape(E, 1, H), fc2_w, fc2_b.reshape(E, 1, D))


@functools.lru_cache(maxsize=None)
def _make_combine_gather():
    @functools.partial(
        pl.kernel,
        out_type=(jax.ShapeDtypeStruct((T, D2), jnp.float32),
                  jax.ShapeDtypeStruct((T, D2), jnp.float32)),
        mesh=plsc.VectorSubcoreMesh(core_axis_name="c", subcore_axis_name="s"),
        scratch_types=[
            pltpu.VMEM((_CHUNK,), jnp.int32),
            pltpu.VMEM((_CHUNK,), jnp.int32),
            pltpu.VMEM((_CHUNK, D2), jnp.float32),
            pltpu.VMEM((_CHUNK, D2), jnp.float32),
            pltpu.SemaphoreType.DMA,
            pltpu.SemaphoreType.DMA,
        ],
    )
    def _combine_gather(y_hbm, d0_hbm, d1_hbm, a_hbm, b_hbm,
                        i0_v, i1_v, ra_v, rb_v, s0, s1):
        wid = lax.axis_index("s") * _NC + lax.axis_index("c")
        base = wid * _CHUNK
        pltpu.sync_copy(d0_hbm.at[pl.ds(base, _CHUNK)], i0_v)
        pltpu.sync_copy(d1_hbm.at[pl.ds(base, _CHUNK)], i1_v)
        c0 = pltpu.async_copy(y_hbm.at[i0_v], ra_v, s0)
        c1 = pltpu.async_copy(y_hbm.at[i1_v], rb_v, s1)
        c0.wait()
        pltpu.sync_copy(ra_v, a_hbm.at[pl.ds(base, _CHUNK)])
        c1.wait()
        pltpu.sync_copy(rb_v, b_hbm.at[pl.ds(base, _CHUNK)])

    return _combine_gather


def _epilogue_body(a_ref, b_ref, g_ref, o_ref):
    alo, ahi = _unpack16(a_ref[...])
    blo, bhi = _unpack16(b_ref[...])
    g0 = g_ref[:, 0:1]
    g1 = g_ref[:, 1:2]
    clo = g0 * jnp.exp(alo) + g1 * jnp.exp(blo)
    chi = g0 * jnp.exp(ahi) + g1 * jnp.exp(bhi)
    o_ref[:, :D2] = jnp.log(jnp.where(clo == 0.0, EPS, clo))
    o_ref[:, D2:] = jnp.log(jnp.where(chi == 0.0, EPS, chi))


def _epilogue(a, b, g):
    return pl.pallas_call(
        _epilogue_body,
        grid=(NE,),
        in_specs=[
            pl.BlockSpec((BE, D2), lambda i: (i, 0)),
            pl.BlockSpec((BE, D2), lambda i: (i, 0)),
            pl.BlockSpec((BE, K), lambda i: (i, 0)),
        ],
        out_specs=pl.BlockSpec((BE, D), lambda i: (i, 0)),
        out_shape=jax.ShapeDtypeStruct((T, D), jnp.float32),
        compiler_params=pltpu.CompilerParams(
            dimension_semantics=("parallel",),
        ),
    )(a, b, g)


@jax.jit
def kernel(x, cond, w_gate, fc1_w, fc1_b, fc2_w, fc2_b):
    g, d0, d1, be, xp = _route(x, cond, w_gate)
    x_sorted = _make_dispatch()(xp, d0, d1)
    y_sorted = _ffn(be, x_sorted, fc1_w, fc1_b, fc2_w, fc2_b)
    a, b = _make_combine_gather()(y_sorted, d0, d1)
    return _epilogue(a, b, g)


# speedup vs baseline: 1.1529x; 1.0019x over previous
import functools

import jax
import jax.numpy as jnp
import numpy as np
from jax import lax
from jax.experimental import pallas as pl
from jax.experimental.pallas import tpu as pltpu
from jax.experimental.pallas import tpu_sc as plsc

D = 768
H = 1536
E = 8
T = 2048
K = 2

BT = 1024
NT = T // BT
BE = 512
NE = T // BE
BR = 512
NB_MAX = (T * K) // BR + E - 1
NBC = NB_MAX + 1
P = NBC * BR
EPS = float(np.finfo(np.float64).eps)

_SQRT1_2 = float(1.0 / np.sqrt(2.0))
D2 = D // 2


def _pack16(lo, hi):
    lo32 = lax.bitcast_convert_type(
        lo.astype(jnp.bfloat16).astype(jnp.float32), jnp.uint32) >> 16
    hi32 = lax.bitcast_convert_type(
        hi.astype(jnp.bfloat16).astype(jnp.float32), jnp.uint32) & jnp.uint32(0xFFFF0000)
    return lax.bitcast_convert_type(hi32 | lo32, jnp.float32)


def _unpack16(w):
    u = lax.bitcast_convert_type(w, jnp.uint32)
    hi = lax.bitcast_convert_type(u & jnp.uint32(0xFFFF0000), jnp.float32)
    lo = lax.bitcast_convert_type(u << 16, jnp.float32)
    return lo, hi


def _route_body(x_ref, c_ref, wg_ref, g_out, d0_out, d1_out, be_out, xp_out,
                nbt_out, carry_sc, pref_sc, e12g_sc, off_sc):
    i = pl.program_id(0)
    iota8 = lax.broadcasted_iota(jnp.int32, (BT, E), 1)

    @pl.when(i < NT)
    def _pass1():
        xb = x_ref[...]
        cb = c_ref[...]
        xp_out[...] = _pack16(xb[:, :D2], xb[:, D2:])
        logits = (jnp.dot(xb, wg_ref[:D, :], preferred_element_type=jnp.float32)
                  + jnp.dot(cb, wg_ref[D:, :], preferred_element_type=jnp.float32))
        m1 = jnp.max(logits, axis=1, keepdims=True)
        e1 = jnp.min(jnp.where(logits >= m1, iota8, E), axis=1, keepdims=True)
        oh1 = iota8 == e1
        neg = jnp.where(oh1, -jnp.inf, logits)
        m2 = jnp.max(neg, axis=1, keepdims=True)
        e2 = jnp.min(jnp.where(neg >= m2, iota8, E), axis=1, keepdims=True)
        oh2 = iota8 == e2
        g1 = jax.nn.sigmoid(m1 - m2)
        g2 = 1.0 - g1
        g_out[...] = jnp.concatenate([g1, g2], axis=1)

        onehot = oh1.astype(jnp.float32) + oh2.astype(jnp.float32)

        @pl.when(i == 0)
        def _():
            carry_sc[...] = jnp.zeros((1, E), jnp.float32)

        inc = onehot
        sh = 1
        while sh < BT:
            inc = inc + jnp.concatenate(
                [jnp.zeros((sh, E), jnp.float32), inc[:BT - sh]], axis=0)
            sh *= 2
        pref = (inc - onehot) + carry_sc[...]
        pref_sc[pl.ds(i * BT, BT), :] = pref
        e12g_sc[pl.ds(i * BT, BT), :] = jnp.concatenate(
            [e1.astype(jnp.float32), e2.astype(jnp.float32), g1, g2,
             jnp.zeros((BT, E - 4), jnp.float32)], axis=1)
        counts = carry_sc[...] + jnp.sum(onehot, axis=0, keepdims=True)
        carry_sc[...] = counts

        @pl.when(i == NT - 1)
        def _finalize():
            cblk = jnp.ceil(counts * (1.0 / BR))
            tr = lax.broadcasted_iota(jnp.int32, (E, E), 0)
            tc = lax.broadcasted_iota(jnp.int32, (E, E), 1)
            tri8 = (tr < tc).astype(jnp.float32)
            off_blk = jnp.dot(cblk, tri8, preferred_element_type=jnp.float32)
            off_sc[...] = off_blk * BR
            nb_tot = jnp.sum(cblk)
            iob = lax.broadcasted_iota(jnp.int32, (1, NBC), 1).astype(jnp.float32)
            acc = jnp.zeros((1, NBC), jnp.float32)
            for e in range(E):
                acc += (iob >= off_blk[0, e]).astype(jnp.float32)
            be_out[...] = (acc - 1.0).astype(jnp.int32)
            nbt_out[...] = jnp.full((1, 1), nb_tot, jnp.float32).astype(jnp.int32)

    @pl.when(i >= NT)
    def _pass2():
        j = i - NT
        pref = pref_sc[pl.ds(j * BT, BT), :]
        e12g = e12g_sc[pl.ds(j * BT, BT), :]
        e1 = e12g[:, 0:1].astype(jnp.int32)
        e2 = e12g[:, 1:2].astype(jnp.int32)
        off = off_sc[...]
        pos = off + pref
        oh1 = (iota8 == e1).astype(jnp.float32)
        oh2 = (iota8 == e2).astype(jnp.float32)
        d0 = jnp.sum(oh1 * pos, axis=1)
        d1 = jnp.sum(oh2 * pos, axis=1)
        d0_out[...] = d0.astype(jnp.int32)
        d1_out[...] = d1.astype(jnp.int32)


def _route(x, cond, w_gate):
    g, d0, d1, be, xp, nbt = pl.pallas_call(
        _route_body,
        grid=(2 * NT,),
        in_specs=[
            pl.BlockSpec((BT, D), lambda i: (jnp.minimum(i, NT - 1), 0)),
            pl.BlockSpec((BT, D), lambda i: (jnp.minimum(i, NT - 1), 0)),
            pl.BlockSpec((2 * D, E), lambda i: (0, 0)),
        ],
        out_specs=[
            pl.BlockSpec((BT, K), lambda i: (jnp.minimum(i, NT - 1), 0)),
            pl.BlockSpec((BT,), lambda i: (jnp.maximum(i - NT, 0),)),
            pl.BlockSpec((BT,), lambda i: (jnp.maximum(i - NT, 0),)),
            pl.BlockSpec((1, NBC), lambda i: (0, 0)),
            pl.BlockSpec((BT, D2), lambda i: (jnp.minimum(i, NT - 1), 0)),
            pl.BlockSpec((1, 1), lambda i: (0, 0)),
        ],
        out_shape=[
            jax.ShapeDtypeStruct((T, K), jnp.float32),
            jax.ShapeDtypeStruct((T,), jnp.int32),
            jax.ShapeDtypeStruct((T,), jnp.int32),
            jax.ShapeDtypeStruct((1, NBC), jnp.int32),
            jax.ShapeDtypeStruct((T, D2), jnp.float32),
            jax.ShapeDtypeStruct((1, 1), jnp.int32),
        ],
        scratch_shapes=[
            pltpu.VMEM((1, E), jnp.float32),
            pltpu.VMEM((T, E), jnp.float32),
            pltpu.VMEM((T, E), jnp.float32),
            pltpu.VMEM((1, E), jnp.float32),
        ],
        compiler_params=pltpu.CompilerParams(
            dimension_semantics=("arbitrary",),
        ),
    )(x, cond, w_gate)
    return g, d0, d1, be, xp, nbt


_NC = 2
_NS = 16
_NW = _NC * _NS
_CHUNK = T // _NW


@functools.lru_cache(maxsize=None)
def _make_dispatch():
    @functools.partial(
        pl.kernel,
        out_type=jax.ShapeDtypeStruct((P, D2), jnp.float32),
        mesh=plsc.VectorSubcoreMesh(core_axis_name="c", subcore_axis_name="s"),
        scratch_types=[
            pltpu.VMEM((_CHUNK,), jnp.int32),
            pltpu.VMEM((_CHUNK,), jnp.int32),
            pltpu.VMEM((_CHUNK, D2), jnp.float32),
            pltpu.SemaphoreType.DMA,
            pltpu.SemaphoreType.DMA,
        ],
    )
    def _dispatch(x_hbm, d0_hbm, d1_hbm, out_hbm, i0_v, i1_v, rows_v, s0, s1):
        wid = lax.axis_index("s") * _NC + lax.axis_index("c")
        base = wid * _CHUNK
        pltpu.sync_copy(d0_hbm.at[pl.ds(base, _CHUNK)], i0_v)
        pltpu.sync_copy(d1_hbm.at[pl.ds(base, _CHUNK)], i1_v)
        pltpu.sync_copy(x_hbm.at[pl.ds(base, _CHUNK)], rows_v)
        c0 = pltpu.async_copy(rows_v, out_hbm.at[i0_v], s0)
        c1 = pltpu.async_copy(rows_v, out_hbm.at[i1_v], s1)
        c0.wait()
        c1.wait()

    return _dispatch


def _ffn_body(be_ref, nbt_ref, x_ref, w1_ref, b1_ref, w2_ref, b2_ref, y_ref):
    b = pl.program_id(0)

    @pl.when(b < nbt_ref[0, 0])
    def _():
        xlo, xhi = _unpack16(x_ref[...])
        xb = jnp.concatenate([xlo, xhi], axis=1)
        h = (jnp.dot(xb, w1_ref[0], preferred_element_type=jnp.float32,
                     precision=lax.Precision.DEFAULT) + b1_ref[0])
        h = 0.5 * h * (1.0 + lax.erf(h * _SQRT1_2))
        o = (jnp.dot(h, w2_ref[0], preferred_element_type=jnp.float32,
                     precision=lax.Precision.DEFAULT) + b2_ref[0])
        y_ref[...] = _pack16(o[:, :D2], o[:, D2:])


def _ffn(be, nbt, x_sorted, fc1_w, fc1_b, fc2_w, fc2_b):
    grid_spec = pltpu.PrefetchScalarGridSpec(
        num_scalar_prefetch=2,
        grid=(NBC,),
        in_specs=[
            pl.BlockSpec((BR, D2),
                         lambda b, be, nbt: (jnp.minimum(b, nbt[0, 0] - 1), 0)),
            pl.BlockSpec((1, D, H), lambda b, be, nbt: (be[0, b], 0, 0)),
            pl.BlockSpec((1, 1, H), lambda b, be, nbt: (be[0, b], 0, 0)),
            pl.BlockSpec((1, H, D), lambda b, be, nbt: (be[0, b], 0, 0)),
            pl.BlockSpec((1, 1, D), lambda b, be, nbt: (be[0, b], 0, 0)),
        ],
        out_specs=pl.BlockSpec(
            (BR, D2),
            lambda b, be, nbt: (jnp.where(b < nbt[0, 0], b, NBC - 1), 0)),
    )
    return pl.pallas_call(
        _ffn_body,
        grid_spec=grid_spec,
        out_shape=jax.ShapeDtypeStruct((P, D2), jnp.float32),
        compiler_params=pltpu.CompilerParams(
            dimension_semantics=("arbitrary",),
        ),
    )(be, nbt, x_sorted, fc1_w, fc1_b.reshape(E, 1, H), fc2_w,
      fc2_b.reshape(E, 1, D))


@functools.lru_cache(maxsize=None)
def _make_combine_gather():
    @functools.partial(
        pl.kernel,
        out_type=(jax.ShapeDtypeStruct((T, D2), jnp.float32),
                  jax.ShapeDtypeStruct((T, D2), jnp.float32)),
        mesh=plsc.VectorSubcoreMesh(core_axis_name="c", subcore_axis_name="s"),
        scratch_types=[
            pltpu.VMEM((_CHUNK,), jnp.int32),
            pltpu.VMEM((_CHUNK,), jnp.int32),
            pltpu.VMEM((_CHUNK, D2), jnp.float32),
            pltpu.VMEM((_CHUNK, D2), jnp.float32),
            pltpu.SemaphoreType.DMA,
            pltpu.SemaphoreType.DMA,
        ],
    )
    def _combine_gather(y_hbm, d0_hbm, d1_hbm, a_hbm, b_hbm,
                        i0_v, i1_v, ra_v, rb_v, s0, s1):
        wid = lax.axis_index("s") * _NC + lax.axis_index("c")
        base = wid * _CHUNK
        pltpu.sync_copy(d0_hbm.at[pl.ds(base, _CHUNK)], i0_v)
        pltpu.sync_copy(d1_hbm.at[pl.ds(base, _CHUNK)], i1_v)
        c0 = pltpu.async_copy(y_hbm.at[i0_v], ra_v, s0)
        c1 = pltpu.async_copy(y_hbm.at[i1_v], rb_v, s1)
        c0.wait()
        pltpu.sync_copy(ra_v, a_hbm.at[pl.ds(base, _CHUNK)])
        c1.wait()
        pltpu.sync_copy(rb_v, b_hbm.at[pl.ds(base, _CHUNK)])

    return _combine_gather


def _epilogue_body(a_ref, b_ref, g_ref, o_ref):
    alo, ahi = _unpack16(a_ref[...])
    blo, bhi = _unpack16(b_ref[...])
    g0 = g_ref[:, 0:1]
    g1 = g_ref[:, 1:2]
    clo = g0 * jnp.exp(alo) + g1 * jnp.exp(blo)
    chi = g0 * jnp.exp(ahi) + g1 * jnp.exp(bhi)
    o_ref[:, :D2] = jnp.log(jnp.where(clo == 0.0, EPS, clo))
    o_ref[:, D2:] = jnp.log(jnp.where(chi == 0.0, EPS, chi))


def _epilogue(a, b, g):
    return pl.pallas_call(
        _epilogue_body,
        grid=(NE,),
        in_specs=[
            pl.BlockSpec((BE, D2), lambda i: (i, 0)),
            pl.BlockSpec((BE, D2), lambda i: (i, 0)),
            pl.BlockSpec((BE, K), lambda i: (i, 0)),
        ],
        out_specs=pl.BlockSpec((BE, D), lambda i: (i, 0)),
        out_shape=jax.ShapeDtypeStruct((T, D), jnp.float32),
        compiler_params=pltpu.CompilerParams(
            dimension_semantics=("parallel",),
        ),
    )(a, b, g)


@jax.jit
def kernel(x, cond, w_gate, fc1_w, fc1_b, fc2_w, fc2_b):
    g, d0, d1, be, xp, nbt = _route(x, cond, w_gate)
    x_sorted = _make_dispatch()(xp, d0, d1)
    y_sorted = _ffn(be, nbt, x_sorted, fc1_w, fc1_b, fc2_w, fc2_b)
    a, b = _make_combine_gather()(y_sorted, d0, d1)
    return _epilogue(a, b, g)
